# baseline (speedup 1.0000x reference)
import functools

import jax
import jax.numpy as jnp
from jax import lax
from jax.experimental import pallas as pl
from jax.experimental.pallas import tpu as pltpu
from jax.experimental.pallas import tpu_sc as plsc

_HID = 64
_LANES = 16
_NBUF = 4
_HALF = 2
_TCB = 8


@functools.lru_cache(maxsize=None)
def _build_sc(n_batch: int, seq: int, vocab: int):
  info = plsc.get_sparse_core_info()
  nw = info.num_cores * info.num_subcores
  chunk = seq
  n_chunks = n_batch // nw
  assert n_chunks % _NBUF == 0
  flat = seq * _HID
  mesh = plsc.VectorSubcoreMesh(core_axis_name="c", subcore_axis_name="s")

  @functools.partial(
      pl.kernel,
      mesh=mesh,
      compiler_params=pltpu.CompilerParams(
          use_tc_tiling_on_sc=False, needs_layout_passes=False),
      out_type=jax.ShapeDtypeStruct((n_batch, flat), jnp.float32),
      scratch_types=[
          pltpu.VMEM((seq, _HID), jnp.float32),
          pltpu.VMEM((n_chunks, chunk), jnp.int32),
          pltpu.VMEM((_NBUF, chunk, _HID), jnp.float32),
          pltpu.VMEM((_NBUF, flat), jnp.float32),
      ] + [pltpu.SemaphoreType.DMA] * (2 * _NBUF),
  )
  def emb(idx_hbm, tok_hbm, pos_hbm, out_hbm, pos_v, idx_v, gat, obuf, *sems):
    sem_in = sems[:_NBUF]
    sem_out = sems[_NBUF:]
    wid = lax.axis_index("s") * info.num_cores + lax.axis_index("c")
    w_batch = wid * n_chunks
    pltpu.sync_copy(pos_hbm, pos_v)
    pltpu.sync_copy(idx_hbm.at[pl.ds(w_batch, n_chunks)], idx_v)

    def gather(c, slot):
      return pltpu.make_async_copy(
          tok_hbm.at[idx_v.at[c]], gat.at[slot], sem_in[slot])

    def scatter(c, slot):
      return pltpu.make_async_copy(
          obuf.at[slot], out_hbm.at[w_batch + c], sem_out[slot])

    for b in range(_HALF):
      gather(b, b).start()

    @pl.loop(0, n_chunks, step=_NBUF)
    def _group(g):
      for b in range(_NBUF):
        c = g + b
        cp = c + _HALF
        pslot = (b + _HALF) % _NBUF

        @pl.when(cp < n_chunks)
        def _prefetch():
          gather(cp, pslot).start()

        gather(c, b).wait()

        @pl.when(c >= _NBUF)
        def _retire():
          scatter(c - _NBUF, b).wait()

        @pl.loop(0, chunk, unroll=4)
        def _row(s):
          for hb in range(_HID // _LANES):
            sl = pl.ds(hb * _LANES, _LANES)
            obuf[b, pl.ds(s * _HID + hb * _LANES, _LANES)] = (
                gat[b, s, sl] + pos_v[s, sl])

        scatter(c, b).start()

    for b in range(_NBUF):
      scatter(n_chunks - _NBUF + b, b).wait()

  return emb


def _tc_transpose(z_ref, o_ref):
  v = z_ref[...]
  nb, flat = v.shape
  seq = flat // _HID
  o_ref[...] = jnp.swapaxes(v.reshape(nb, seq, _HID), 1, 2)


@functools.lru_cache(maxsize=None)
def _build_tc(n_batch: int, seq: int):
  flat = seq * _HID
  return pl.pallas_call(
      _tc_transpose,
      grid=(n_batch // _TCB,),
      in_specs=[pl.BlockSpec((_TCB, flat), lambda i: (i, 0))],
      out_specs=pl.BlockSpec((_TCB, _HID, seq), lambda i: (i, 0, 0)),
      out_shape=jax.ShapeDtypeStruct((n_batch, _HID, seq), jnp.float32),
  )


def kernel(x, tok_table, pos_table):
  b, s = x.shape
  vocab, hid = tok_table.shape
  z = _build_sc(b, s, vocab)(x.astype(jnp.int32), tok_table, pos_table)
  out_t = _build_tc(b, s)(z)
  return jnp.swapaxes(out_t, 1, 2)


# speedup vs baseline: 1.7641x; 1.7641x over previous
import functools

import jax
import jax.numpy as jnp
from jax import lax
from jax.experimental import pallas as pl
from jax.experimental.pallas import tpu as pltpu
from jax.experimental.pallas import tpu_sc as plsc

_HID = 64
_LANES = 16
_NBUF = 4
_HALF = 2
_TPAD = 1


@functools.lru_cache(maxsize=None)
def _build(n_batch: int, seq: int, vocab: int):
  info = plsc.get_sparse_core_info()
  nw = info.num_cores * info.num_subcores
  chunk = seq
  n_chunks = n_batch // nw
  assert n_chunks % _NBUF == 0
  mesh = plsc.VectorSubcoreMesh(core_axis_name="c", subcore_axis_name="s")

  @functools.partial(
      pl.kernel,
      mesh=mesh,
      compiler_params=pltpu.CompilerParams(
          use_tc_tiling_on_sc=False, needs_layout_passes=False),
      out_type=jax.ShapeDtypeStruct((n_batch, _HID, seq), jnp.float32),
      scratch_types=[
          pltpu.VMEM((seq, _HID), jnp.float32),
          pltpu.VMEM((n_chunks, chunk), jnp.int32),
          pltpu.VMEM((_NBUF, chunk, _HID), jnp.float32),
          pltpu.VMEM((_NBUF, _HID, seq + _TPAD), jnp.float32),
      ] + [pltpu.SemaphoreType.DMA] * (2 * _NBUF),
  )
  def emb(idx_hbm, tok_hbm, pos_hbm, out_hbm, pos_v, idx_v, gat, obuf, *sems):
    sem_in = sems[:_NBUF]
    sem_out = sems[_NBUF:]
    wid = lax.axis_index("s") * info.num_cores + lax.axis_index("c")
    w_batch = wid * n_chunks
    pltpu.sync_copy(pos_hbm, pos_v)
    pltpu.sync_copy(idx_hbm.at[pl.ds(w_batch, n_chunks)], idx_v)

    iota = lax.iota(jnp.int32, _LANES)

    def gather(c, slot):
      return pltpu.make_async_copy(
          tok_hbm.at[idx_v.at[c]], gat.at[slot], sem_in[slot])

    def scatter(c, slot):
      return pltpu.make_async_copy(
          obuf.at[slot, :, pl.ds(0, seq)], out_hbm.at[w_batch + c],
          sem_out[slot])

    for b in range(_HALF):
      gather(b, b).start()

    @pl.loop(0, n_chunks, step=_NBUF)
    def _group(g):
      for b in range(_NBUF):
        c = g + b
        cp = c + _HALF
        pslot = (b + _HALF) % _NBUF

        @pl.when(cp < n_chunks)
        def _prefetch():
          gather(cp, pslot).start()

        gather(c, b).wait()

        @pl.when(c >= _NBUF)
        def _retire():
          scatter(c - _NBUF, b).wait()

        @pl.loop(0, chunk, unroll=4)
        def _row(s):
          s_vec = jnp.full((_LANES,), 0, jnp.int32) + s
          for hb in range(_HID // _LANES):
            sl = pl.ds(hb * _LANES, _LANES)
            y = gat[b, s, sl] + pos_v[s, sl]
            plsc.store_scatter(obuf.at[b], [iota + hb * _LANES, s_vec], y)

        scatter(c, b).start()

    for b in range(_NBUF):
      scatter(n_chunks - _NBUF + b, b).wait()

  return emb


def kernel(x, tok_table, pos_table):
  b, s = x.shape
  vocab, hid = tok_table.shape
  out_t = _build(b, s, vocab)(x.astype(jnp.int32), tok_table, pos_table)
  return jnp.swapaxes(out_t, 1, 2)
